# baseline (speedup 1.0000x reference)
import functools

import jax
import jax.numpy as jnp
from jax import lax
from jax.experimental import pallas as pl
from jax.experimental.pallas import tpu as pltpu
from jax.experimental.pallas import tpu_sc as plsc


def _sc_accumulate(x, pos16, ei, npad, n0, chunk):
  n_cores = 2
  n_sub = 16
  d_feat = x.shape[1]
  rows_per_tile = npad // n_sub
  strips = rows_per_tile // chunk

  mesh = plsc.VectorSubcoreMesh(core_axis_name="c", subcore_axis_name="s")

  @functools.partial(
      pl.kernel,
      mesh=mesh,
      compiler_params=pltpu.CompilerParams(use_tc_tiling_on_sc=False),
      out_type=(
          jax.ShapeDtypeStruct((n_cores, npad, d_feat), jnp.float32),
          jax.ShapeDtypeStruct((n_cores, npad, 16), jnp.float32),
          jax.ShapeDtypeStruct((n_cores, npad, 16), jnp.float32),
      ),
      scratch_types=[
          pltpu.VMEM((4, chunk), jnp.int32),
          pltpu.VMEM((4, chunk), jnp.int32),
          pltpu.VMEM((2, chunk, d_feat), jnp.float32),
          pltpu.VMEM((2, chunk, 16), jnp.float32),
          pltpu.VMEM((chunk, 16), jnp.float32),
          pltpu.SemaphoreType.DMA,
          pltpu.SemaphoreType.DMA,
          pltpu.SemaphoreType.DMA,
          pltpu.SemaphoreType.DMA,
          pltpu.SemaphoreType.DMA,
          pltpu.SemaphoreType.DMA,
          pltpu.SemaphoreType.DMA,
          pltpu.SemaphoreType.DMA,
          pltpu.VMEM_SHARED((npad, d_feat), jnp.float32),
          pltpu.VMEM_SHARED((npad, 16), jnp.float32),
          pltpu.VMEM_SHARED((npad, 16), jnp.float32),
      ],
  )
  def sc_kernel(x_hbm, p_hbm, ei_hbm, agg_out, ns_out, dg_out,
                sidx, didx, xrows, prows, ones16, gsem0, gsem1, ssem0, ssem1,
                isem0, isem1, isem2, isem3, agg_sh, ns_sh, dg_sh):
    c = lax.axis_index("c")
    s = lax.axis_index("s")
    gsems = (gsem0, gsem1)
    ssems = (ssem0, ssem1)
    isems = (isem0, isem1, isem2, isem3)

    onehot3 = jnp.where(lax.iota(jnp.int32, 16) == 3, 1.0, 0.0)

    @pl.loop(0, chunk)
    def _(i):
      for j in range(d_feat // 16):
        xrows[0, i, pl.ds(j * 16, 16)] = jnp.zeros((16,), jnp.float32)
      prows[0, i, :] = jnp.zeros((16,), jnp.float32)
      ones16[i, :] = onehot3

    @pl.loop(0, strips)
    def _(k):
      base = s * rows_per_tile + k * chunk
      pltpu.sync_copy(xrows.at[0], agg_sh.at[pl.ds(base, chunk)])
      pltpu.sync_copy(prows.at[0], ns_sh.at[pl.ds(base, chunk)])
      pltpu.sync_copy(prows.at[0], dg_sh.at[pl.ds(base, chunk)])

    tail = rows_per_tile - strips * chunk
    if tail:
      base = s * rows_per_tile + strips * chunk
      pltpu.sync_copy(xrows.at[0, pl.ds(0, tail)],
                      agg_sh.at[pl.ds(base, tail)])
      pltpu.sync_copy(prows.at[0, pl.ds(0, tail)],
                      ns_sh.at[pl.ds(base, tail)])
      pltpu.sync_copy(prows.at[0, pl.ds(0, tail)],
                      dg_sh.at[pl.ds(base, tail)])

    plsc.subcore_barrier()

    def issue(pairs):
      for args in pairs:
        pltpu.async_copy(*args)

    def wait(pairs):
      for args in pairs:
        pltpu.make_async_copy(*args).wait()

    n_it = n0
    row0 = (c * n_sub + s) * n_it

    def idx_pair(i, q):
      r = (row0 + i) * chunk
      return (
          (ei_hbm.at[0, pl.ds(r, chunk)], sidx.at[q], isems[q]),
          (ei_hbm.at[1, pl.ds(r, chunk)], didx.at[q], isems[q]),
      )

    def gather_pair(b, q):
      return (
          (x_hbm.at[sidx.at[q]], xrows.at[b], gsems[b]),
          (p_hbm.at[didx.at[q]], prows.at[b], gsems[b]),
      )

    def scatter_trip(b, q):
      return (
          (xrows.at[b], agg_sh.at[didx.at[q]], ssems[b]),
          (prows.at[b], ns_sh.at[sidx.at[q]], ssems[b]),
          (ones16, dg_sh.at[didx.at[q]], ssems[b]),
      )

    def issue_add(trips):
      for args in trips:
        pltpu.async_copy(*args, add=True)

    for q in range(4):
      issue(idx_pair(q, q))
    wait(idx_pair(0, 0))
    issue(gather_pair(0, 0))

    @pl.loop(0, n_it, step=4)
    def _(i0):
      for j in range(4):
        i = i0 + j
        b = j % 2
        q = j
        qn = (j + 1) % 4
        qp = (j + 3) % 4
        wait(gather_pair(b, q))
        issue_add(scatter_trip(b, q))
        if j == 0:

          @pl.when(i0 > 0)
          def _():
            wait(scatter_trip(1 - b, qp))
            issue(idx_pair(i + 3, qp))
        else:
          wait(scatter_trip(1 - b, qp))

          @pl.when(i + 3 < n_it)
          def _():
            issue(idx_pair(i + 3, qp))

        @pl.when(i + 1 < n_it)
        def _():
          wait(idx_pair(i + 1, qn))
          issue(gather_pair(1 - b, qn))

    wait(scatter_trip(1, 3))
    plsc.subcore_barrier()

    base = s * rows_per_tile
    pltpu.sync_copy(agg_sh.at[pl.ds(base, rows_per_tile)],
                    agg_out.at[c, pl.ds(base, rows_per_tile)])
    pltpu.sync_copy(ns_sh.at[pl.ds(base, rows_per_tile)],
                    ns_out.at[c, pl.ds(base, rows_per_tile)])
    pltpu.sync_copy(dg_sh.at[pl.ds(base, rows_per_tile)],
                    dg_out.at[c, pl.ds(base, rows_per_tile)])

  return sc_kernel(x, pos16, ei)


def _tc_body(x_ref, p_ref, agg_ref, ns_ref, dg_ref, w_ref, b_ref, o_ref):
  agg2 = agg_ref[...].astype(jnp.float32)
  agg = agg2[0] + agg2[1]
  dg2 = dg_ref[...]
  deg = jnp.maximum(dg2[0, :, 3:4] + dg2[1, :, 3:4], 1.0)
  ns2 = ns_ref[...]
  ns = ns2[0] + ns2[1]
  cnt = jnp.maximum(ns[:, 3:4], 1.0)
  d = p_ref[...] - ns / cnt
  col = lax.broadcasted_iota(jnp.int32, d.shape, 1)
  d = jnp.where(col < 3, d, 0.0)
  curv = jnp.sqrt(jnp.sum(d * d, axis=1, keepdims=True))
  h = (x_ref[...] + agg / deg) * (1.0 + curv)
  o_ref[...] = (
      jnp.dot(h, w_ref[...], preferred_element_type=jnp.float32) + b_ref[...]
  )


def kernel(x, positions, edge_index, W, b):
  N, D = x.shape
  E = edge_index.shape[1]
  chunk = 80
  n_sub = 16
  rows_total = -(-E // chunk)
  n0 = ((-(-rows_total // 32) + 3) // 4) * 4
  epad = 2 * n_sub * n0 * chunk
  npad = ((N + 64) // 64) * 64
  dummy = N

  ei = jnp.asarray(edge_index, jnp.int32)
  pad = epad - E
  if pad:
    pad_block = jnp.concatenate(
        [jnp.zeros((1, pad), jnp.int32), jnp.full((1, pad), dummy, jnp.int32)])
    ei = jnp.concatenate([ei, pad_block], axis=1)

  pos16 = jnp.concatenate(
      [positions, jnp.ones((N, 1), jnp.float32),
       jnp.zeros((N, 12), jnp.float32)], axis=1)
  pos16 = jnp.concatenate(
      [pos16, jnp.zeros((npad - N, 16), jnp.float32)])

  agg_out, ns_out, dg_out = _sc_accumulate(x, pos16, ei, npad, n0, chunk)

  blk = 1000
  grid = (N // blk,)
  out = pl.pallas_call(
      _tc_body,
      grid=grid,
      in_specs=[
          pl.BlockSpec((blk, D), lambda i: (i, 0)),
          pl.BlockSpec((blk, 16), lambda i: (i, 0)),
          pl.BlockSpec((2, blk, D), lambda i: (0, i, 0)),
          pl.BlockSpec((2, blk, 16), lambda i: (0, i, 0)),
          pl.BlockSpec((2, blk, 16), lambda i: (0, i, 0)),
          pl.BlockSpec((D, D), lambda i: (0, 0)),
          pl.BlockSpec((1, D), lambda i: (0, 0)),
      ],
      out_specs=pl.BlockSpec((blk, D), lambda i: (i, 0)),
      out_shape=jax.ShapeDtypeStruct((N, D), jnp.float32),
  )(x, pos16, agg_out, ns_out, dg_out, W, b.reshape(1, D))
  return out


# speedup vs baseline: 1.5655x; 1.5655x over previous
import functools

import jax
import jax.numpy as jnp
from jax import lax
from jax.experimental import pallas as pl
from jax.experimental.pallas import tpu as pltpu
from jax.experimental.pallas import tpu_sc as plsc


def _sc_accumulate(x, pos16, ei, npad, n0, chunk):
  n_cores = 2
  n_sub = 16
  d_feat = x.shape[1]
  rows_per_tile = npad // n_sub
  strips = rows_per_tile // chunk

  mesh = plsc.VectorSubcoreMesh(core_axis_name="c", subcore_axis_name="s")

  @functools.partial(
      pl.kernel,
      mesh=mesh,
      compiler_params=pltpu.CompilerParams(use_tc_tiling_on_sc=False),
      out_type=(
          jax.ShapeDtypeStruct((n_cores, npad, d_feat), jnp.float32),
          jax.ShapeDtypeStruct((n_cores, npad, 16), jnp.float32),
          jax.ShapeDtypeStruct((n_cores, npad, 16), jnp.float32),
      ),
      scratch_types=[
          pltpu.VMEM((2, chunk), jnp.int32),
          pltpu.VMEM((2, chunk), jnp.int32),
          pltpu.VMEM((2, chunk, d_feat), jnp.float32),
          pltpu.VMEM((2, chunk, 16), jnp.float32),
          pltpu.VMEM((chunk, 16), jnp.float32),
          pltpu.SemaphoreType.DMA,
          pltpu.SemaphoreType.DMA,
          pltpu.SemaphoreType.DMA,
          pltpu.SemaphoreType.DMA,
          pltpu.VMEM_SHARED((npad, d_feat), jnp.float32),
          pltpu.VMEM_SHARED((npad, 16), jnp.float32),
          pltpu.VMEM_SHARED((npad, 16), jnp.float32),
      ],
  )
  def sc_kernel(x_hbm, p_hbm, ei_hbm, agg_out, ns_out, dg_out,
                sidx, didx, xrows, prows, ones16, gsem0, gsem1, isem0, isem1,
                agg_sh, ns_sh, dg_sh):
    c = lax.axis_index("c")
    s = lax.axis_index("s")
    gsems = (gsem0, gsem1)
    isems = (isem0, isem1)

    onehot3 = jnp.where(lax.iota(jnp.int32, 16) == 3, 1.0, 0.0)

    @pl.loop(0, chunk)
    def _(i):
      for j in range(d_feat // 16):
        xrows[0, i, pl.ds(j * 16, 16)] = jnp.zeros((16,), jnp.float32)
      prows[0, i, :] = jnp.zeros((16,), jnp.float32)
      ones16[i, :] = onehot3

    @pl.loop(0, strips)
    def _(k):
      base = s * rows_per_tile + k * chunk
      pltpu.sync_copy(xrows.at[0], agg_sh.at[pl.ds(base, chunk)])
      pltpu.sync_copy(prows.at[0], ns_sh.at[pl.ds(base, chunk)])
      pltpu.sync_copy(prows.at[0], dg_sh.at[pl.ds(base, chunk)])

    tail = rows_per_tile - strips * chunk
    if tail:
      base = s * rows_per_tile + strips * chunk
      pltpu.sync_copy(xrows.at[0, pl.ds(0, tail)],
                      agg_sh.at[pl.ds(base, tail)])
      pltpu.sync_copy(prows.at[0, pl.ds(0, tail)],
                      ns_sh.at[pl.ds(base, tail)])
      pltpu.sync_copy(prows.at[0, pl.ds(0, tail)],
                      dg_sh.at[pl.ds(base, tail)])

    plsc.subcore_barrier()

    def issue(pairs):
      for args in pairs:
        pltpu.async_copy(*args)

    def wait(pairs):
      for args in pairs:
        pltpu.make_async_copy(*args).wait()

    n_it = n0
    row0 = (c * n_sub + s) * n_it

    def idx_pair(i, b):
      r = (row0 + i) * chunk
      return (
          (ei_hbm.at[0, pl.ds(r, chunk)], sidx.at[b], isems[b]),
          (ei_hbm.at[1, pl.ds(r, chunk)], didx.at[b], isems[b]),
      )

    def gather_pair(b):
      return (
          (x_hbm.at[sidx.at[b]], xrows.at[b], gsems[b]),
          (p_hbm.at[didx.at[b]], prows.at[b], gsems[b]),
      )

    def do_scatters(b):
      pltpu.sync_copy(xrows.at[b], agg_sh.at[didx.at[b]], add=True)
      pltpu.sync_copy(prows.at[b], ns_sh.at[sidx.at[b]], add=True)
      pltpu.sync_copy(ones16, dg_sh.at[didx.at[b]], add=True)

    issue(idx_pair(0, 0))
    wait(idx_pair(0, 0))
    issue(idx_pair(1, 1))
    issue(gather_pair(0))

    @pl.loop(0, n_it, step=2)
    def _(i0):
      for b in range(2):
        i = i0 + b
        wait(gather_pair(b))

        @pl.when(i + 1 < n_it)
        def _():
          wait(idx_pair(i + 1, 1 - b))
          issue(gather_pair(1 - b))

        do_scatters(b)

        @pl.when(i + 2 < n_it)
        def _():
          issue(idx_pair(i + 2, b))

    plsc.subcore_barrier()

    base = s * rows_per_tile
    pltpu.sync_copy(agg_sh.at[pl.ds(base, rows_per_tile)],
                    agg_out.at[c, pl.ds(base, rows_per_tile)])
    pltpu.sync_copy(ns_sh.at[pl.ds(base, rows_per_tile)],
                    ns_out.at[c, pl.ds(base, rows_per_tile)])
    pltpu.sync_copy(dg_sh.at[pl.ds(base, rows_per_tile)],
                    dg_out.at[c, pl.ds(base, rows_per_tile)])

  return sc_kernel(x, pos16, ei)


def _tc_body(x_ref, p_ref, agg_ref, ns_ref, dg_ref, w_ref, b_ref, o_ref):
  agg2 = agg_ref[...].astype(jnp.float32)
  agg = agg2[0] + agg2[1]
  dg2 = dg_ref[...]
  deg = jnp.maximum(dg2[0, :, 3:4] + dg2[1, :, 3:4], 1.0)
  ns2 = ns_ref[...]
  ns = ns2[0] + ns2[1]
  cnt = jnp.maximum(ns[:, 3:4], 1.0)
  d = p_ref[...] - ns / cnt
  col = lax.broadcasted_iota(jnp.int32, d.shape, 1)
  d = jnp.where(col < 3, d, 0.0)
  curv = jnp.sqrt(jnp.sum(d * d, axis=1, keepdims=True))
  h = (x_ref[...] + agg / deg) * (1.0 + curv)
  o_ref[...] = (
      jnp.dot(h, w_ref[...], preferred_element_type=jnp.float32) + b_ref[...]
  )


def kernel(x, positions, edge_index, W, b):
  N, D = x.shape
  E = edge_index.shape[1]
  chunk = 80
  n_sub = 16
  rows_total = -(-E // chunk)
  n0 = -(-rows_total // 32)
  n0 += n0 % 2
  epad = 2 * n_sub * n0 * chunk
  npad = ((N + 64) // 64) * 64
  dummy = N

  ei = jnp.asarray(edge_index, jnp.int32)
  pad = epad - E
  if pad:
    pad_block = jnp.concatenate(
        [jnp.zeros((1, pad), jnp.int32), jnp.full((1, pad), dummy, jnp.int32)])
    ei = jnp.concatenate([ei, pad_block], axis=1)

  pos16 = jnp.concatenate(
      [positions, jnp.ones((N, 1), jnp.float32),
       jnp.zeros((N, 12), jnp.float32)], axis=1)
  pos16 = jnp.concatenate(
      [pos16, jnp.zeros((npad - N, 16), jnp.float32)])

  agg_out, ns_out, dg_out = _sc_accumulate(x, pos16, ei, npad, n0, chunk)

  blk = 1000
  grid = (N // blk,)
  out = pl.pallas_call(
      _tc_body,
      grid=grid,
      in_specs=[
          pl.BlockSpec((blk, D), lambda i: (i, 0)),
          pl.BlockSpec((blk, 16), lambda i: (i, 0)),
          pl.BlockSpec((2, blk, D), lambda i: (0, i, 0)),
          pl.BlockSpec((2, blk, 16), lambda i: (0, i, 0)),
          pl.BlockSpec((2, blk, 16), lambda i: (0, i, 0)),
          pl.BlockSpec((D, D), lambda i: (0, 0)),
          pl.BlockSpec((1, D), lambda i: (0, 0)),
      ],
      out_specs=pl.BlockSpec((blk, D), lambda i: (i, 0)),
      out_shape=jax.ShapeDtypeStruct((N, D), jnp.float32),
  )(x, pos16, agg_out, ns_out, dg_out, W, b.reshape(1, D))
  return out


# speedup vs baseline: 2.1511x; 1.3740x over previous
import functools

import jax
import jax.numpy as jnp
from jax import lax
from jax.experimental import pallas as pl
from jax.experimental.pallas import tpu as pltpu
from jax.experimental.pallas import tpu_sc as plsc


def _sc_accumulate(x, pos16, ei, npad, n0, chunk):
  n_cores = 2
  n_sub = 16
  d_feat = x.shape[1]
  rows_per_tile = npad // n_sub
  strips = rows_per_tile // chunk

  mesh = plsc.VectorSubcoreMesh(core_axis_name="c", subcore_axis_name="s")

  @functools.partial(
      pl.kernel,
      mesh=mesh,
      compiler_params=pltpu.CompilerParams(use_tc_tiling_on_sc=False),
      out_type=(
          jax.ShapeDtypeStruct((n_cores, npad, d_feat), jnp.float32),
          jax.ShapeDtypeStruct((n_cores, npad, 16), jnp.float32),
          jax.ShapeDtypeStruct((n_cores, npad, 16), jnp.float32),
      ),
      scratch_types=[
          pltpu.VMEM((2, chunk), jnp.int32),
          pltpu.VMEM((2, chunk), jnp.int32),
          pltpu.VMEM((2, chunk, d_feat), jnp.float32),
          pltpu.VMEM((2, chunk, 16), jnp.float32),
          pltpu.VMEM((chunk, 16), jnp.float32),
          pltpu.SemaphoreType.DMA,
          pltpu.SemaphoreType.DMA,
          pltpu.SemaphoreType.DMA,
          pltpu.SemaphoreType.DMA,
          pltpu.VMEM_SHARED((npad, d_feat), jnp.float32),
          pltpu.VMEM_SHARED((npad, 16), jnp.float32),
          pltpu.VMEM_SHARED((npad, 16), jnp.float32),
      ],
  )
  def sc_kernel(x_hbm, p_hbm, ei_hbm, agg_out, ns_out, dg_out,
                sidx, didx, xrows, prows, ones16, gsem0, gsem1, isem0, isem1,
                agg_sh, ns_sh, dg_sh):
    c = lax.axis_index("c")
    s = lax.axis_index("s")
    gsems = (gsem0, gsem1)
    isems = (isem0, isem1)

    onehot3 = jnp.where(lax.iota(jnp.int32, 16) == 3, 1.0, 0.0)

    @pl.loop(0, chunk)
    def _(i):
      for j in range(d_feat // 16):
        xrows[0, i, pl.ds(j * 16, 16)] = jnp.zeros((16,), jnp.float32)
      prows[0, i, :] = jnp.zeros((16,), jnp.float32)
      ones16[i, :] = onehot3

    @pl.loop(0, strips)
    def _(k):
      base = s * rows_per_tile + k * chunk
      pltpu.sync_copy(xrows.at[0], agg_sh.at[pl.ds(base, chunk)])
      pltpu.sync_copy(prows.at[0], ns_sh.at[pl.ds(base, chunk)])
      pltpu.sync_copy(prows.at[0], dg_sh.at[pl.ds(base, chunk)])

    tail = rows_per_tile - strips * chunk
    if tail:
      base = s * rows_per_tile + strips * chunk
      pltpu.sync_copy(xrows.at[0, pl.ds(0, tail)],
                      agg_sh.at[pl.ds(base, tail)])
      pltpu.sync_copy(prows.at[0, pl.ds(0, tail)],
                      ns_sh.at[pl.ds(base, tail)])
      pltpu.sync_copy(prows.at[0, pl.ds(0, tail)],
                      dg_sh.at[pl.ds(base, tail)])

    plsc.subcore_barrier()

    def issue(pairs):
      for args in pairs:
        pltpu.async_copy(*args)

    def wait(pairs):
      for args in pairs:
        pltpu.make_async_copy(*args).wait()

    n_it = n0
    row0 = (c * n_sub + s) * n_it

    def idx_pair(i, b):
      r = (row0 + i) * chunk
      return (
          (ei_hbm.at[0, pl.ds(r, chunk)], sidx.at[b], isems[b]),
          (ei_hbm.at[1, pl.ds(r, chunk)], didx.at[b], isems[b]),
      )

    def gather_pair(b):
      return (
          (x_hbm.at[sidx.at[b]], xrows.at[b], gsems[b]),
          (p_hbm.at[didx.at[b]], prows.at[b], gsems[b]),
      )

    def do_scatters(b):
      pltpu.sync_copy(xrows.at[b], agg_sh.at[didx.at[b]], add=True)
      pltpu.sync_copy(prows.at[b], ns_sh.at[sidx.at[b]], add=True)
      pltpu.sync_copy(ones16, dg_sh.at[didx.at[b]], add=True)

    issue(idx_pair(0, 0))
    wait(idx_pair(0, 0))
    issue(idx_pair(1, 1))
    issue(gather_pair(0))

    n_even = n_it - (n_it % 2)

    @pl.loop(0, n_even, step=2)
    def _(i0):
      for b in range(2):
        i = i0 + b
        wait(gather_pair(b))

        @pl.when(i + 1 < n_it)
        def _():
          wait(idx_pair(i + 1, 1 - b))
          issue(gather_pair(1 - b))

        do_scatters(b)

        @pl.when(i + 2 < n_it)
        def _():
          issue(idx_pair(i + 2, b))

    if n_it % 2:
      wait(gather_pair((n_it - 1) % 2))
      do_scatters((n_it - 1) % 2)

    plsc.subcore_barrier()

    base = s * rows_per_tile
    pltpu.sync_copy(agg_sh.at[pl.ds(base, rows_per_tile)],
                    agg_out.at[c, pl.ds(base, rows_per_tile)])
    pltpu.sync_copy(ns_sh.at[pl.ds(base, rows_per_tile)],
                    ns_out.at[c, pl.ds(base, rows_per_tile)])
    pltpu.sync_copy(dg_sh.at[pl.ds(base, rows_per_tile)],
                    dg_out.at[c, pl.ds(base, rows_per_tile)])

  return sc_kernel(x, pos16, ei)


def _tc_body(x_ref, p_ref, agg_ref, ns_ref, dg_ref, w_ref, b_ref, o_ref):
  agg2 = agg_ref[...].astype(jnp.float32)
  agg = agg2[0] + agg2[1]
  dg2 = dg_ref[...]
  deg = jnp.maximum(dg2[0, :, 3:4] + dg2[1, :, 3:4], 1.0)
  ns2 = ns_ref[...]
  ns = ns2[0] + ns2[1]
  cnt = jnp.maximum(ns[:, 3:4], 1.0)
  d = p_ref[...] - ns / cnt
  col = lax.broadcasted_iota(jnp.int32, d.shape, 1)
  d = jnp.where(col < 3, d, 0.0)
  curv = jnp.sqrt(jnp.sum(d * d, axis=1, keepdims=True))
  h = (x_ref[...] + agg / deg) * (1.0 + curv)
  o_ref[...] = (
      jnp.dot(h, w_ref[...], preferred_element_type=jnp.float32) + b_ref[...]
  )


def kernel(x, positions, edge_index, W, b):
  N, D = x.shape
  E = edge_index.shape[1]
  chunk = 80
  n_sub = 16
  rows_total = -(-E // chunk)
  n0 = max(2, -(-rows_total // 32))
  epad = 2 * n_sub * n0 * chunk
  npad = ((N + 64) // 64) * 64
  dummy = N

  ei = jnp.asarray(edge_index, jnp.int32)
  pad = epad - E
  if pad:
    pad_block = jnp.concatenate(
        [jnp.zeros((1, pad), jnp.int32), jnp.full((1, pad), dummy, jnp.int32)])
    ei = jnp.concatenate([ei, pad_block], axis=1)

  pos16 = jnp.concatenate(
      [positions, jnp.ones((N, 1), jnp.float32),
       jnp.zeros((N, 12), jnp.float32)], axis=1)
  pos16 = jnp.concatenate(
      [pos16, jnp.zeros((npad - N, 16), jnp.float32)])

  agg_out, ns_out, dg_out = _sc_accumulate(x, pos16, ei, npad, n0, chunk)

  blk = 1000
  grid = (N // blk,)
  out = pl.pallas_call(
      _tc_body,
      grid=grid,
      in_specs=[
          pl.BlockSpec((blk, D), lambda i: (i, 0)),
          pl.BlockSpec((blk, 16), lambda i: (i, 0)),
          pl.BlockSpec((2, blk, D), lambda i: (0, i, 0)),
          pl.BlockSpec((2, blk, 16), lambda i: (0, i, 0)),
          pl.BlockSpec((2, blk, 16), lambda i: (0, i, 0)),
          pl.BlockSpec((D, D), lambda i: (0, 0)),
          pl.BlockSpec((1, D), lambda i: (0, 0)),
      ],
      out_specs=pl.BlockSpec((blk, D), lambda i: (i, 0)),
      out_shape=jax.ShapeDtypeStruct((N, D), jnp.float32),
  )(x, pos16, agg_out, ns_out, dg_out, W, b.reshape(1, D))
  return out
